# Initial kernel scaffold; baseline (speedup 1.0000x reference)
import jax
import jax.numpy as jnp
from jax.experimental import pallas as pl


def kernel(x1, edge_index, edge_attr, batch, W1, as1, ad1, We1, ae1, b1, W2, as2, ad2, We2, ae2, b2):
    raise NotImplementedError("write your pallas kernel here")


# speedup vs baseline: 1.0917x; 1.0917x over previous
import functools

import jax
import jax.numpy as jnp
from jax.experimental import pallas as pl
from jax.experimental.pallas import tpu as pltpu

N = 10000
E = 160000
G = 128


def _dense_body(x_ref, w_ref, a3_ref, h_ref, al_ref):
    x = x_ref[...]
    h_ref[...] = jnp.dot(x, w_ref[...], preferred_element_type=jnp.float32)
    al_ref[...] = jnp.dot(x, a3_ref[...], preferred_element_type=jnp.float32)


def _mat_body(x_ref, a3_ref, al_ref):
    al_ref[...] = jnp.dot(x_ref[...], a3_ref[...],
                          preferred_element_type=jnp.float32)


def _mat_block(x, A3, bn):
    M, F = x.shape
    J = A3.shape[1]
    return pl.pallas_call(
        _mat_body,
        grid=(M // bn,),
        in_specs=[
            pl.BlockSpec((bn, F), lambda i: (i, 0)),
            pl.BlockSpec((F, J), lambda i: (0, 0)),
        ],
        out_specs=pl.BlockSpec((bn, J), lambda i: (i, 0)),
        out_shape=jax.ShapeDtypeStruct((M, J), jnp.float32),
    )(x, A3)


def _dense_block(x, W, A3, bn):
    M, F = x.shape
    K = W.shape[1]
    J = A3.shape[1]
    grid = (M // bn,)
    return pl.pallas_call(
        _dense_body,
        grid=grid,
        in_specs=[
            pl.BlockSpec((bn, F), lambda i: (i, 0)),
            pl.BlockSpec((F, K), lambda i: (0, 0)),
            pl.BlockSpec((F, J), lambda i: (0, 0)),
        ],
        out_specs=[
            pl.BlockSpec((bn, K), lambda i: (i, 0)),
            pl.BlockSpec((bn, J), lambda i: (i, 0)),
        ],
        out_shape=[
            jax.ShapeDtypeStruct((M, K), jnp.float32),
            jax.ShapeDtypeStruct((M, J), jnp.float32),
        ],
    )(x, W, A3)


def kernel(x1, edge_index, edge_attr, batch, W1, as1, ad1, We1, ae1, b1,
           W2, as2, ad2, We2, ae2, b2):
    src, dst = edge_index[0], edge_index[1]
    loop = jnp.arange(N, dtype=src.dtype)
    src_f = jnp.concatenate([src, loop])
    dst_f = jnp.concatenate([dst, loop])
    ei = jnp.stack([src_f, dst_f])

    def layer(x, W, a_s, a_d, We, a_e, bias, H, C):
        F = x.shape[1]
        As = (W.reshape(F, H, C) * a_s).sum(-1)
        Ad = (W.reshape(F, H, C) * a_d).sum(-1)
        Ae = (We.reshape(-1, H, C) * a_e).sum(-1)
        A3 = jnp.concatenate([As, Ad], axis=1)
        h, al = _dense_block(x, W, A3, 1000)
        asrc, adst = al[:, :H], al[:, H:]
        ae_edges = _mat_block(edge_attr, Ae, 1000)
        mean_attr = jnp.mean(edge_attr, axis=0, keepdims=True)
        ae_self = jnp.broadcast_to(mean_attr @ Ae, (N, H))
        ae_full = jnp.concatenate([ae_edges, ae_self], 0)
        a = asrc[src_f] + adst[dst_f] + ae_full
        a = jax.nn.leaky_relu(a, 0.2)
        p = jnp.exp(a)
        denom = jax.ops.segment_sum(p, dst_f, num_segments=N)
        hm = h.reshape(N, H, C)
        msg = hm[src_f] * p[:, :, None]
        out_un = jax.ops.segment_sum(msg, dst_f, num_segments=N)
        out = out_un / (denom[:, :, None] + 1e-16)
        alpha = p / (denom[dst_f] + 1e-16)
        return out.reshape(N, H * C) + bias, alpha

    h1, _ = layer(x1, W1, as1, ad1, We1, ae1, b1, 10, 16)
    h1 = jax.nn.elu(h1)
    h2, w2 = layer(h1, W2, as2, ad2, We2, ae2, b2, 1, 16)
    h2 = jax.nn.elu(h2)
    x_mean = jax.ops.segment_max(h2, batch, num_segments=G)
    return (x_mean, ei, w2[:, None] if w2.ndim == 1 else w2)
